# baseline (speedup 1.0000x reference)
import functools

import jax
import jax.numpy as jnp
from jax import lax
from jax.experimental import pallas as pl
from jax.experimental.pallas import tpu as pltpu
from jax.experimental.pallas import tpu_sc as plsc

_CHUNK = 10000
_CHAINS = 10
_LOG2E = 1.4426950408889634

_V = 100000
_N = 128


def _slab_copy(x_hbm, scr, sem, i, c):
    return pltpu.make_async_copy(
        x_hbm.at[pl.ds(i * c, c), :], scr.at[pl.ds(i * c, c), :], sem.at[i]
    )


def _out_copy(scr, probs_hbm, sem, i, c):
    return pltpu.make_async_copy(
        scr.at[pl.ds(i * c, c), :], probs_hbm.at[pl.ds(i * c, c), :], sem
    )


def _softmax_body(x_hbm, probs_hbm, scr, m_ref, s_ref, insem, outsem):
    v, n = scr.shape
    c = _CHUNK
    ns = v // c
    k = _CHAINS
    d = c // k

    for j in range(ns):
        _slab_copy(x_hbm, scr, insem, j, c).start()

    m_ref[...] = jnp.full((1, n), -jnp.inf, jnp.float32)
    s_ref[...] = jnp.zeros((1, n), jnp.float32)

    def stats_step(i, carry):
        _slab_copy(x_hbm, scr, insem, i, c).wait()
        xr = scr[pl.ds(i * c, c), :].reshape(k, d, n)
        run_m = m_ref[...]
        pm = jnp.max(xr, axis=1)
        cmax = jnp.max(pm, axis=0, keepdims=True)
        nmax = jnp.maximum(run_m, cmax)
        bm = nmax * _LOG2E
        ps = jnp.sum(jnp.exp2(xr * _LOG2E - bm[None]), axis=1)
        csum = jnp.sum(ps, axis=0, keepdims=True)
        s_ref[...] = s_ref[...] * jnp.exp2(run_m * _LOG2E - bm) + csum
        m_ref[...] = nmax
        return carry

    lax.fori_loop(0, ns, stats_step, 0, unroll=False)

    b = m_ref[...] * _LOG2E + jnp.log2(s_ref[...])

    def write_step(i, carry):
        x = scr[pl.ds(i * c, c), :]
        scr[pl.ds(i * c, c), :] = jnp.exp2(x * _LOG2E - b)
        _out_copy(scr, probs_hbm, outsem, i, c).start()
        return carry

    lax.fori_loop(0, ns, write_step, 0, unroll=False)

    def drain_step(i, carry):
        _out_copy(scr, probs_hbm, outsem, i, c).wait()
        return carry

    lax.fori_loop(0, ns, drain_step, 0, unroll=False)


def _tc_softmax(x_t):
    v, n = x_t.shape
    return pl.pallas_call(
        _softmax_body,
        in_specs=[pl.BlockSpec(memory_space=pl.ANY)],
        out_specs=pl.BlockSpec(memory_space=pl.ANY),
        out_shape=jax.ShapeDtypeStruct((v, n), jnp.float32),
        scratch_shapes=[
            pltpu.VMEM((v, n), jnp.float32),
            pltpu.VMEM((1, n), jnp.float32),
            pltpu.VMEM((1, n), jnp.float32),
            pltpu.SemaphoreType.DMA((v // _CHUNK,)),
            pltpu.SemaphoreType.DMA,
        ],
    )(x_t)


_SC_CH = 200
_SC_UNROLL = 10
_SC_NCH = _V // 2 // _SC_CH


def _sc_argmax_body(
    x_hbm, rs_hbm, pm_hbm, pa_hbm,
    buf, rbuf, stg_m, stg_a, shm, sha, cmb_m, cmb_a, out_m, out_a, insem, rsem,
):
    cid = lax.axis_index("c")
    sid = lax.axis_index("s")
    n = _N
    ng = n // 16
    ch = _SC_CH
    nch_core = _SC_NCH
    max_ci = (nch_core + 15) // 16
    base = cid * (_V // 2)

    def gchunk(ci):
        return cid * nch_core + sid + 16 * ci

    def in_copy(slot, ci):
        row = base + (sid + 16 * ci) * ch
        return pltpu.make_async_copy(
            x_hbm.at[pl.ds(row, ch), :], buf.at[slot], insem.at[slot]
        )

    def rs_copy(slot, ci):
        return pltpu.make_async_copy(
            rs_hbm.at[pl.ds(gchunk(ci) * 16, 16)], rbuf.at[slot], rsem.at[slot]
        )

    def valid(ci):
        return sid + 16 * ci < nch_core

    for pslot in (0, 1):

        @pl.when(valid(pslot))
        def _prime(_s=pslot):
            in_copy(_s, _s).start()
            rs_copy(_s, _s).start()

    m = [jnp.full((16,), -jnp.inf, jnp.float32) for _ in range(ng)]
    a = [jnp.zeros((16,), jnp.int32) for _ in range(ng)]

    def make_row_block(slot):
        def row_block(rb, carry):
            cm = list(carry[:ng])
            ca = list(carry[ng : 2 * ng])
            rvec = carry[2 * ng]
            r = rb * _SC_UNROLL
            for u in range(_SC_UNROLL):
                ruv = rvec + jnp.full((16,), u, jnp.int32)
                for g in range(ng):
                    x = buf[slot, r + u, pl.ds(g * 16, 16)]
                    gt = x > cm[g]
                    ca[g] = jnp.where(gt, ruv, ca[g])
                    cm[g] = jnp.where(gt, x, cm[g])
            rvec = rvec + jnp.full((16,), _SC_UNROLL, jnp.int32)
            return tuple(cm) + tuple(ca) + (rvec,)

        return row_block

    for ci in range(max_ci - 1):
        slot = ci % 2
        in_copy(slot, ci).wait()
        rs_copy(slot, ci).wait()
        rvec0 = rbuf[slot, pl.ds(0, 16)]
        carry = lax.fori_loop(
            0,
            ch // _SC_UNROLL,
            make_row_block(slot),
            tuple(m) + tuple(a) + (rvec0,),
            unroll=False,
        )
        m = list(carry[:ng])
        a = list(carry[ng : 2 * ng])
        if ci + 2 < max_ci:

            @pl.when(valid(ci + 2))
            def _pf(_slot=slot, _ci=ci):
                in_copy(_slot, _ci + 2).start()
                rs_copy(_slot, _ci + 2).start()

    for g in range(ng):
        stg_m[0, pl.ds(g * 16, 16)] = m[g]
        stg_a[0, pl.ds(g * 16, 16)] = a[g]

    @pl.when(valid(max_ci - 1))
    def _last():
        ci = max_ci - 1
        slot = ci % 2
        in_copy(slot, ci).wait()
        rs_copy(slot, ci).wait()
        rvec0 = rbuf[slot, pl.ds(0, 16)]
        lm = [stg_m[0, pl.ds(g * 16, 16)] for g in range(ng)]
        la = [stg_a[0, pl.ds(g * 16, 16)] for g in range(ng)]
        carry = lax.fori_loop(
            0,
            ch // _SC_UNROLL,
            make_row_block(slot),
            tuple(lm) + tuple(la) + (rvec0,),
            unroll=False,
        )
        for g in range(ng):
            stg_m[0, pl.ds(g * 16, 16)] = carry[g]
            stg_a[0, pl.ds(g * 16, 16)] = carry[ng + g]

    pltpu.sync_copy(stg_m.at[0], shm.at[sid])
    pltpu.sync_copy(stg_a.at[0], sha.at[sid])
    plsc.subcore_barrier()

    @pl.when(sid == 0)
    def _combine():
        pltpu.sync_copy(shm, cmb_m)
        pltpu.sync_copy(sha, cmb_a)
        for g in range(ng):
            fm = cmb_m[0, pl.ds(g * 16, 16)]
            fa = cmb_a[0, pl.ds(g * 16, 16)]
            for w in range(1, 16):
                xm = cmb_m[w, pl.ds(g * 16, 16)]
                xa = cmb_a[w, pl.ds(g * 16, 16)]
                gt = jnp.logical_or(xm > fm, jnp.logical_and(xm == fm, xa < fa))
                fa = jnp.where(gt, xa, fa)
                fm = jnp.where(gt, xm, fm)
            out_m[0, pl.ds(g * 16, 16)] = fm
            out_a[0, pl.ds(g * 16, 16)] = fa
        pltpu.sync_copy(out_m.at[0], pm_hbm.at[pl.ds(cid * n, n)])
        pltpu.sync_copy(out_a.at[0], pa_hbm.at[pl.ds(cid * n, n)])


def _sc_argmax(x_t, rowstart):
    mesh = plsc.VectorSubcoreMesh(core_axis_name="c", subcore_axis_name="s")
    kern = functools.partial(
        pl.kernel,
        mesh=mesh,
        out_type=[
            jax.ShapeDtypeStruct((2 * _N,), jnp.float32),
            jax.ShapeDtypeStruct((2 * _N,), jnp.int32),
        ],
        scratch_types=[
            pltpu.VMEM((2, _SC_CH, _N), jnp.float32),
            pltpu.VMEM((2, 16), jnp.int32),
            pltpu.VMEM((1, _N), jnp.float32),
            pltpu.VMEM((1, _N), jnp.int32),
            pltpu.VMEM_SHARED((16, _N), jnp.float32),
            pltpu.VMEM_SHARED((16, _N), jnp.int32),
            pltpu.VMEM((16, _N), jnp.float32),
            pltpu.VMEM((16, _N), jnp.int32),
            pltpu.VMEM((1, _N), jnp.float32),
            pltpu.VMEM((1, _N), jnp.int32),
            pltpu.SemaphoreType.DMA((2,)),
            pltpu.SemaphoreType.DMA((2,)),
        ],
    )(_sc_argmax_body)
    return kern(x_t, rowstart)


def _combine_body(pm_ref, pa_ref, ids_ref):
    n = _N
    m0 = pm_ref[0, :]
    m1 = pm_ref[1, :]
    a0 = pa_ref[0, :]
    a1 = pa_ref[1, :]
    gt = m1 > m0
    ids_ref[...] = jnp.where(gt, a1, a0).reshape(1, n)


def _tc_combine(pm, pa):
    n = _N
    ids = pl.pallas_call(
        _combine_body,
        in_specs=[
            pl.BlockSpec((2, n), lambda: (0, 0)),
            pl.BlockSpec((2, n), lambda: (0, 0)),
        ],
        out_specs=pl.BlockSpec((1, n), lambda: (0, 0)),
        out_shape=jax.ShapeDtypeStruct((1, n), jnp.int32),
    )(pm.reshape(2, n), pa.reshape(2, n))
    return ids.reshape(n)


def kernel(logits):
    n, v = logits.shape
    x_t = logits.T
    nch = 2 * _SC_NCH
    rowstart = jnp.repeat(jnp.arange(nch, dtype=jnp.int32) * _SC_CH, 16)
    probs_t = _tc_softmax(x_t)
    pm, pa = _sc_argmax(x_t, rowstart)
    ids = _tc_combine(pm, pa)
    return (ids, probs_t.T)


# speedup vs baseline: 2.2241x; 2.2241x over previous
import jax
import jax.numpy as jnp
from jax import lax
from jax.experimental import pallas as pl
from jax.experimental.pallas import tpu as pltpu

_CHUNK = 10000
_CHAINS = 10
_LOG2E = 1.4426950408889634


def _slab_copy(x_hbm, scr, sem, i, c):
    return pltpu.make_async_copy(
        x_hbm.at[pl.ds(i * c, c), :], scr.at[pl.ds(i * c, c), :], sem.at[i]
    )


def _out_copy(scr, probs_hbm, sem, i, c):
    return pltpu.make_async_copy(
        scr.at[pl.ds(i * c, c), :], probs_hbm.at[pl.ds(i * c, c), :], sem
    )


def _sampler_body(x_hbm, ids_ref, probs_hbm, scr, m_ref, s_ref, a_ref, insem, outsem):
    v, n = scr.shape
    c = _CHUNK
    ns = v // c
    k = _CHAINS
    d = c // k

    for j in range(ns):
        _slab_copy(x_hbm, scr, insem, j, c).start()

    m_ref[...] = jnp.full((1, n), -jnp.inf, jnp.float32)
    s_ref[...] = jnp.zeros((1, n), jnp.float32)
    a_ref[...] = jnp.zeros((1, n), jnp.int32)

    def stats_step(i, carry):
        _slab_copy(x_hbm, scr, insem, i, c).wait()
        xr = scr[pl.ds(i * c, c), :].reshape(k, d, n)
        run_m = m_ref[...]
        pm = jnp.max(xr, axis=1)
        cmax = jnp.max(pm, axis=0, keepdims=True)
        row = lax.broadcasted_iota(jnp.int32, (k, d, n), 1)
        parg = jnp.min(jnp.where(xr == cmax[None], row, v), axis=1)
        offs = lax.broadcasted_iota(jnp.int32, (k, 1), 0) * d
        carg = jnp.min(parg + offs, axis=0, keepdims=True) + i * c
        nmax = jnp.maximum(run_m, cmax)
        bm = nmax * _LOG2E
        ps = jnp.sum(jnp.exp2(xr * _LOG2E - bm[None]), axis=1)
        csum = jnp.sum(ps, axis=0, keepdims=True)
        s_ref[...] = s_ref[...] * jnp.exp2(run_m * _LOG2E - bm) + csum
        a_ref[...] = jnp.where(cmax > run_m, carg, a_ref[...])
        m_ref[...] = nmax
        return carry

    lax.fori_loop(0, ns, stats_step, 0, unroll=False)

    b = m_ref[...] * _LOG2E + jnp.log2(s_ref[...])

    cw = c // 4

    def write_step(i, carry):
        x = scr[pl.ds(i * cw, cw), :]
        scr[pl.ds(i * cw, cw), :] = jnp.exp2(x * _LOG2E - b)
        _out_copy(scr, probs_hbm, outsem, i, cw).start()
        return carry

    lax.fori_loop(0, 4 * ns, write_step, 0, unroll=False)

    def drain_step(i, carry):
        _out_copy(scr, probs_hbm, outsem, i, cw).wait()
        return carry

    lax.fori_loop(0, 4 * ns, drain_step, 0, unroll=False)
    ids_ref[...] = a_ref[...]


def kernel(logits):
    n, v = logits.shape
    x_t = logits.T
    ids, probs_t = pl.pallas_call(
        _sampler_body,
        in_specs=[pl.BlockSpec(memory_space=pl.ANY)],
        out_specs=[
            pl.BlockSpec((1, n), lambda: (0, 0)),
            pl.BlockSpec(memory_space=pl.ANY),
        ],
        out_shape=[
            jax.ShapeDtypeStruct((1, n), jnp.int32),
            jax.ShapeDtypeStruct((v, n), jnp.float32),
        ],
        scratch_shapes=[
            pltpu.VMEM((v, n), jnp.float32),
            pltpu.VMEM((1, n), jnp.float32),
            pltpu.VMEM((1, n), jnp.float32),
            pltpu.VMEM((1, n), jnp.int32),
            pltpu.SemaphoreType.DMA((v // _CHUNK,)),
            pltpu.SemaphoreType.DMA,
        ],
    )(x_t)
    return (ids.reshape(n), probs_t.T)


# speedup vs baseline: 2.3442x; 1.0540x over previous
import jax
import jax.numpy as jnp
from jax import lax
from jax.experimental import pallas as pl
from jax.experimental.pallas import tpu as pltpu

_CHUNK = 5000
_CHAINS = 5
_LOG2E = 1.4426950408889634


def _slab_copy(x_hbm, scr, sem, i, c):
    return pltpu.make_async_copy(
        x_hbm.at[pl.ds(i * c, c), :], scr.at[pl.ds(i * c, c), :], sem.at[i]
    )


def _out_copy(scr, probs_hbm, sem, i, c):
    return pltpu.make_async_copy(
        scr.at[pl.ds(i * c, c), :], probs_hbm.at[pl.ds(i * c, c), :], sem
    )


def _sampler_body(x_hbm, ids_ref, probs_hbm, scr, m_ref, s_ref, a_ref, insem, outsem):
    v, n = scr.shape
    c = _CHUNK
    ns = v // c
    k = _CHAINS
    d = c // k

    for j in range(ns):
        _slab_copy(x_hbm, scr, insem, j, c).start()

    m_ref[...] = jnp.full((1, n), -jnp.inf, jnp.float32)
    s_ref[...] = jnp.zeros((1, n), jnp.float32)
    a_ref[...] = jnp.zeros((1, n), jnp.int32)

    def stats_step(i, carry):
        _slab_copy(x_hbm, scr, insem, i, c).wait()
        xr = scr[pl.ds(i * c, c), :].reshape(k, d, n)
        run_m = m_ref[...]
        pm = jnp.max(xr, axis=1)
        cmax = jnp.max(pm, axis=0, keepdims=True)
        row = lax.broadcasted_iota(jnp.int32, (k, d, n), 1)
        parg = jnp.min(jnp.where(xr == cmax[None], row, v), axis=1)
        offs = lax.broadcasted_iota(jnp.int32, (k, 1), 0) * d
        carg = jnp.min(parg + offs, axis=0, keepdims=True) + i * c
        nmax = jnp.maximum(run_m, cmax)
        bm = nmax * _LOG2E
        ps = jnp.sum(jnp.exp2(xr * _LOG2E - bm[None]), axis=1)
        csum = jnp.sum(ps, axis=0, keepdims=True)
        s_ref[...] = s_ref[...] * jnp.exp2(run_m * _LOG2E - bm) + csum
        a_ref[...] = jnp.where(cmax > run_m, carg, a_ref[...])
        m_ref[...] = nmax
        return carry

    lax.fori_loop(0, ns, stats_step, 0, unroll=False)

    b = m_ref[...] * _LOG2E + jnp.log2(s_ref[...])

    cw = c // 4

    def write_step(i, carry):
        x = scr[pl.ds(i * cw, cw), :]
        scr[pl.ds(i * cw, cw), :] = jnp.exp2(x * _LOG2E - b)
        _out_copy(scr, probs_hbm, outsem, i, cw).start()
        return carry

    lax.fori_loop(0, 4 * ns, write_step, 0, unroll=False)

    def drain_step(i, carry):
        _out_copy(scr, probs_hbm, outsem, i, cw).wait()
        return carry

    lax.fori_loop(0, 4 * ns, drain_step, 0, unroll=False)
    ids_ref[...] = a_ref[...]


def kernel(logits):
    n, v = logits.shape
    x_t = logits.T
    ids, probs_t = pl.pallas_call(
        _sampler_body,
        in_specs=[pl.BlockSpec(memory_space=pl.ANY)],
        out_specs=[
            pl.BlockSpec((1, n), lambda: (0, 0)),
            pl.BlockSpec(memory_space=pl.ANY),
        ],
        out_shape=[
            jax.ShapeDtypeStruct((1, n), jnp.int32),
            jax.ShapeDtypeStruct((v, n), jnp.float32),
        ],
        scratch_shapes=[
            pltpu.VMEM((v, n), jnp.float32),
            pltpu.VMEM((1, n), jnp.float32),
            pltpu.VMEM((1, n), jnp.float32),
            pltpu.VMEM((1, n), jnp.int32),
            pltpu.SemaphoreType.DMA((v // _CHUNK,)),
            pltpu.SemaphoreType.DMA,
        ],
    )(x_t)
    return (ids.reshape(n), probs_t.T)
